# baseline (speedup 1.0000x reference)
import jax
import jax.numpy as jnp
from jax import lax
from jax.experimental import pallas as pl
from jax.experimental.pallas import tpu as pltpu
from jax.experimental.pallas import tpu_sc as plsc

_L = 16


def _sc_body(x_hbm, out_hbm, rows_v, res_v):
    cid = lax.axis_index("c")
    sid = lax.axis_index("s")

    @pl.when(jnp.logical_and(cid == 0, sid == 0))
    def _():
        pltpu.sync_copy(x_hbm.at[pl.ds(0, 8)], rows_v)
        v1 = rows_v[1, pl.ds(0, _L)]
        v2 = rows_v[2, pl.ds(0, _L)]
        neq = jnp.logical_or(v1 != v1, v2 != v2)
        lane = lax.iota(jnp.int32, _L)
        ans = jnp.where(jnp.logical_and(neq, lane == 0), 1.0, 0.0)
        res_v[...] = ans.astype(jnp.float32)
        pltpu.sync_copy(res_v, out_hbm)


@jax.jit
def _sc_gather_compare(x):
    mesh = plsc.VectorSubcoreMesh(core_axis_name="c", subcore_axis_name="s")
    out = pl.kernel(
        _sc_body,
        out_type=jax.ShapeDtypeStruct((_L,), jnp.float32),
        mesh=mesh,
        scratch_types=[
            pltpu.VMEM((8, 64), jnp.float32),
            pltpu.VMEM((_L,), jnp.float32),
        ],
    )(x)
    return out[0]


def kernel(x):
    return _sc_gather_compare(x)


# speedup vs baseline: 1.0404x; 1.0404x over previous
import jax
import jax.numpy as jnp
from jax import lax
from jax.experimental import pallas as pl
from jax.experimental.pallas import tpu as pltpu
from jax.experimental.pallas import tpu_sc as plsc

_L = 16


def _sc_body(x_hbm, out_hbm, rows_v, res_v):
    cid = lax.axis_index("c")
    sid = lax.axis_index("s")

    @pl.when(jnp.logical_and(cid == 0, sid == 0))
    def _():
        pltpu.sync_copy(x_hbm.at[pl.ds(0, 8)], rows_v)
        v1 = rows_v[1, pl.ds(0, _L)]
        v2 = rows_v[2, pl.ds(0, _L)]
        neq = jnp.logical_or(v1 != v1, v2 != v2)
        lane = lax.iota(jnp.int32, _L)
        ans = jnp.where(jnp.logical_and(neq, lane == 0), 1.0, 0.0)
        res_v[...] = ans.astype(jnp.float32)
        pltpu.sync_copy(res_v, out_hbm)


@jax.jit
def _sc_gather_compare(x):
    mesh = plsc.VectorSubcoreMesh(
        core_axis_name="c", subcore_axis_name="s", num_cores=1
    )
    out = pl.kernel(
        _sc_body,
        out_type=jax.ShapeDtypeStruct((_L,), jnp.float32),
        mesh=mesh,
        scratch_types=[
            pltpu.VMEM((8, 64), jnp.float32),
            pltpu.VMEM((_L,), jnp.float32),
        ],
    )(x)
    return out[0]


def kernel(x):
    return _sc_gather_compare(x)


# speedup vs baseline: 1.0554x; 1.0144x over previous
import jax
import jax.numpy as jnp
from jax import lax
from jax.experimental import pallas as pl
from jax.experimental.pallas import tpu as pltpu
from jax.experimental.pallas import tpu_sc as plsc


def _scs_body(x_hbm, out_hbm, buf, res):
    cid = lax.axis_index("c")

    @pl.when(cid == 0)
    def _():
        pltpu.sync_copy(x_hbm.at[pl.ds(0, 8)], buf)
        a = buf[1, 0]
        b = buf[2, 0]
        neq = jnp.logical_or(a != a, b != b)
        res[0] = jnp.where(neq, 1.0, 0.0).astype(jnp.float32)
        pltpu.sync_copy(res, out_hbm)


@jax.jit
def _sc_gather_compare(x):
    mesh = plsc.ScalarSubcoreMesh(axis_name="c", num_cores=1)
    out = pl.kernel(
        _scs_body,
        out_type=jax.ShapeDtypeStruct((1,), jnp.float32),
        mesh=mesh,
        scratch_types=[
            pltpu.SMEM((8, 64), jnp.float32),
            pltpu.SMEM((1,), jnp.float32),
        ],
    )(x)
    return out[0]


def kernel(x):
    return _sc_gather_compare(x)
